# Initial kernel scaffold; baseline (speedup 1.0000x reference)
import jax
import jax.numpy as jnp
from jax.experimental import pallas as pl


def kernel(x, edge_index, pre_v, pre_g, pre_b, w1_v, w1_g, post_v, post_g, post_b):
    raise NotImplementedError("write your pallas kernel here")


# speedup vs baseline: 6.6558x; 6.6558x over previous
import functools

import jax
import jax.numpy as jnp
from jax import lax
from jax.experimental import pallas as pl
from jax.experimental.pallas import tpu as pltpu
from jax.experimental.pallas import tpu_sc as plsc

NN = 10000
EE = 160000
DD = 256
HH = 256
OO = 128
NB = 4
ALPHA_C = 0.1
THETA_C = 0.5

F32 = jnp.float32
_HIGH = lax.Precision.HIGHEST

_MESH = plsc.VectorSubcoreMesh(core_axis_name="c", subcore_axis_name="s")
NCORE = 2
NSUB = 16
HALF = HH // 2
ROWS_PT = NN // NSUB


_DEG_K = 40
_DEG_PT = EE // (NCORE * NSUB)
_DEG_CH = _DEG_PT // _DEG_K


def _deg_body(dst_hbm, counts_hbm, acc, ones_v, idx_v, zero_v):
    cid = lax.axis_index("c")
    sid = lax.axis_index("s")

    for off in (0, 16, 24):
        ones_v[pl.ds(off, 16)] = jnp.ones((16,), F32)
    for r in range(40):
        zero_v[pl.ds(r * 16, 16)] = jnp.zeros((16,), F32)

    @pl.when(sid < NSUB - 1)
    def _():
        pltpu.sync_copy(zero_v, acc.at[pl.ds(sid * 640, 640)])

    @pl.when(sid == NSUB - 1)
    def _():
        pltpu.sync_copy(zero_v.at[pl.ds(0, 400)], acc.at[pl.ds(9600, 400)])

    plsc.subcore_barrier()

    base = (cid * NSUB + sid) * _DEG_PT

    def chunk(k, carry):
        eb = base + k * _DEG_K
        pltpu.sync_copy(dst_hbm.at[pl.ds(eb, _DEG_K)], idx_v)
        pltpu.sync_copy(ones_v.at[pl.ds(0, _DEG_K)], acc.at[idx_v], add=True)
        return carry

    lax.fori_loop(0, _DEG_CH, chunk, 0)
    plsc.subcore_barrier()

    @pl.when(sid < NSUB - 1)
    def _():
        pltpu.sync_copy(acc.at[pl.ds(sid * 640, 640)], zero_v)
        pltpu.sync_copy(zero_v, counts_hbm.at[pl.ds(cid * NN + sid * 640, 640)])

    @pl.when(sid == NSUB - 1)
    def _():
        pltpu.sync_copy(acc.at[pl.ds(9600, 400)], zero_v.at[pl.ds(0, 400)])
        pltpu.sync_copy(zero_v.at[pl.ds(0, 400)],
                        counts_hbm.at[pl.ds(cid * NN + 9600, 400)])


_deg_call = pl.kernel(
    _deg_body,
    out_type=jax.ShapeDtypeStruct((NCORE * NN,), F32),
    mesh=_MESH,
    scratch_types=[
        pltpu.VMEM_SHARED((NN,), F32),
        pltpu.VMEM((40,), F32),
        pltpu.VMEM((_DEG_K,), jnp.int32),
        pltpu.VMEM((640,), F32),
    ],
)


_SEG_K = 80
_SEG_PT = EE // NSUB
_SEG_CH = _SEG_PT // _SEG_K
_ZR = 80


def _seg_rows(sid):
    base = sid * 640
    return base


def _seg_body(xs0_hbm, xs1_hbm, src_hbm, dst_hbm, s0_hbm, s1_hbm,
              acc, src_v, dst_v, rows_v, zero_v, sem):
    cid = lax.axis_index("c")
    sid = lax.axis_index("s")

    def zrow(r, carry):
        for cc in range(HALF // 16):
            zero_v[r, pl.ds(cc * 16, 16)] = jnp.zeros((16,), F32)
        return carry

    lax.fori_loop(0, _ZR, zrow, 0)

    @pl.when(sid < NSUB - 1)
    def _():
        for j in range(640 // _ZR):
            r0 = pl.multiple_of(sid * 640 + j * _ZR, 8)
            pltpu.sync_copy(zero_v, acc.at[pl.ds(r0, _ZR)])

    @pl.when(sid == NSUB - 1)
    def _():
        for j in range(400 // _ZR):
            pltpu.sync_copy(zero_v, acc.at[pl.ds(9600 + j * _ZR, _ZR)])

    plsc.subcore_barrier()

    base = sid * _SEG_PT

    def chunk(k, carry):
        eb = base + k * _SEG_K
        pltpu.sync_copy(src_hbm.at[pl.ds(eb, _SEG_K)], src_v)
        pltpu.sync_copy(dst_hbm.at[pl.ds(eb, _SEG_K)], dst_v)

        @pl.when(cid == 0)
        def _():
            pltpu.async_copy(xs0_hbm.at[src_v], rows_v, sem).wait()

        @pl.when(cid == 1)
        def _():
            pltpu.async_copy(xs1_hbm.at[src_v], rows_v, sem).wait()

        pltpu.sync_copy(rows_v, acc.at[dst_v], add=True)
        return carry

    lax.fori_loop(0, _SEG_CH, chunk, 0)
    plsc.subcore_barrier()

    def copy_out(r0):
        pltpu.sync_copy(acc.at[pl.ds(r0, _ZR)], zero_v)

        @pl.when(cid == 0)
        def _():
            pltpu.sync_copy(zero_v, s0_hbm.at[pl.ds(r0, _ZR)])

        @pl.when(cid == 1)
        def _():
            pltpu.sync_copy(zero_v, s1_hbm.at[pl.ds(r0, _ZR)])

    @pl.when(sid < NSUB - 1)
    def _():
        for j in range(640 // _ZR):
            copy_out(pl.multiple_of(sid * 640 + j * _ZR, 8))

    @pl.when(sid == NSUB - 1)
    def _():
        for j in range(400 // _ZR):
            copy_out(9600 + j * _ZR)


_seg_call = pl.kernel(
    _seg_body,
    out_type=(jax.ShapeDtypeStruct((NN, HALF), F32),
              jax.ShapeDtypeStruct((NN, HALF), F32)),
    mesh=_MESH,
    scratch_types=[
        pltpu.VMEM_SHARED((NN, HALF), F32),
        pltpu.VMEM((_SEG_K,), jnp.int32),
        pltpu.VMEM((_SEG_K,), jnp.int32),
        pltpu.VMEM((_SEG_K, HALF), F32),
        pltpu.VMEM((_ZR, HALF), F32),
        pltpu.SemaphoreType.DMA,
    ],
)


_RB = 1000
_GRID = NN // _RB


def _l2n(x):
    n = jnp.sqrt(jnp.sum(x * x, axis=1, keepdims=True))
    return x / jnp.maximum(n, 1e-12)


def _wn_scale(v_ref, g_ref):
    v = v_ref[...]
    g = g_ref[...]
    nr = jnp.sqrt(jnp.sum(v * v, axis=1))
    return g[0] / nr


def _pre_body(x_ref, cnt_ref, pv_ref, pg_ref, pb_ref,
              x0_ref, dis_ref, xs0_ref, xs1_ref):
    scale = _wn_scale(pv_ref, pg_ref)
    h = jnp.dot(x_ref[...], pv_ref[...].T, precision=_HIGH) * scale[None, :]
    h = h + pb_ref[...][0][None, :]
    x0_ref[...] = h
    deg = 1.0 + cnt_ref[...][:, 0] + cnt_ref[...][:, 1]
    dis = 1.0 / jnp.sqrt(deg)
    dis_ref[...] = dis[:, None]
    xs = dis[:, None] * _l2n(h)
    xs0_ref[...] = xs[:, :HALF]
    xs1_ref[...] = xs[:, HALF:]


def _pre_call(x, counts_t, pre_v, pre_g, pre_b):
    return pl.pallas_call(
        _pre_body,
        grid=(_GRID,),
        in_specs=[
            pl.BlockSpec((_RB, DD), lambda i: (i, 0)),
            pl.BlockSpec((_RB, 2), lambda i: (i, 0)),
            pl.BlockSpec((HH, DD), lambda i: (0, 0)),
            pl.BlockSpec((1, HH), lambda i: (0, 0)),
            pl.BlockSpec((1, HH), lambda i: (0, 0)),
        ],
        out_specs=[
            pl.BlockSpec((_RB, HH), lambda i: (i, 0)),
            pl.BlockSpec((_RB, 1), lambda i: (i, 0)),
            pl.BlockSpec((_RB, HALF), lambda i: (i, 0)),
            pl.BlockSpec((_RB, HALF), lambda i: (i, 0)),
        ],
        out_shape=[
            jax.ShapeDtypeStruct((NN, HH), F32),
            jax.ShapeDtypeStruct((NN, 1), F32),
            jax.ShapeDtypeStruct((NN, HALF), F32),
            jax.ShapeDtypeStruct((NN, HALF), F32),
        ],
    )(x, counts_t, pre_v, pre_g, pre_b)


def _blk_body(s0_ref, s1_ref, xs0_ref, xs1_ref, x0_ref, dis_ref,
              wv_ref, wg_ref, beta_ref, *out_refs, last):
    s = jnp.concatenate([s0_ref[...], s1_ref[...]], axis=1)
    xs = jnp.concatenate([xs0_ref[...], xs1_ref[...]], axis=1)
    dis = dis_ref[...]
    beta = beta_ref[...][0, 0]
    agg = dis * (s + xs)
    hh = (1.0 - ALPHA_C) * agg + ALPHA_C * x0_ref[...]
    scale = _wn_scale(wv_ref, wg_ref)
    mat = jnp.dot(hh * scale[None, :], wv_ref[...], precision=_HIGH)
    xc = (1.0 - beta) * hh + beta * mat
    xc = jnp.where(xc >= 0, xc, 0.01 * xc)
    if last:
        out_refs[0][...] = xc
    else:
        xsn = dis * _l2n(xc)
        out_refs[0][...] = xsn[:, :HALF]
        out_refs[1][...] = xsn[:, HALF:]


def _blk_call(s0, s1, xs0, xs1, x0, dis, wv, wg, beta, last):
    if last:
        out_specs = [pl.BlockSpec((_RB, HH), lambda i: (i, 0))]
        out_shape = [jax.ShapeDtypeStruct((NN, HH), F32)]
    else:
        out_specs = [pl.BlockSpec((_RB, HALF), lambda i: (i, 0)),
                     pl.BlockSpec((_RB, HALF), lambda i: (i, 0))]
        out_shape = [jax.ShapeDtypeStruct((NN, HALF), F32),
                     jax.ShapeDtypeStruct((NN, HALF), F32)]
    return pl.pallas_call(
        functools.partial(_blk_body, last=last),
        grid=(_GRID,),
        in_specs=[
            pl.BlockSpec((_RB, HALF), lambda i: (i, 0)),
            pl.BlockSpec((_RB, HALF), lambda i: (i, 0)),
            pl.BlockSpec((_RB, HALF), lambda i: (i, 0)),
            pl.BlockSpec((_RB, HALF), lambda i: (i, 0)),
            pl.BlockSpec((_RB, HH), lambda i: (i, 0)),
            pl.BlockSpec((_RB, 1), lambda i: (i, 0)),
            pl.BlockSpec((HH, HH), lambda i: (0, 0)),
            pl.BlockSpec((1, HH), lambda i: (0, 0)),
            pl.BlockSpec((1, 1), lambda i: (0, 0)),
        ],
        out_specs=out_specs,
        out_shape=out_shape,
    )(s0, s1, xs0, xs1, x0, dis, wv, wg, beta)


def _post_body(xc_ref, pv_ref, pg_ref, pb_ref, out_ref):
    scale = _wn_scale(pv_ref, pg_ref)
    y = jnp.dot(xc_ref[...], pv_ref[...].T, precision=_HIGH) * scale[None, :]
    y = y + pb_ref[...][0][None, :]
    out_ref[...] = _l2n(y)


def _post_call(xc, post_v, post_g, post_b):
    return pl.pallas_call(
        _post_body,
        grid=(_GRID,),
        in_specs=[
            pl.BlockSpec((_RB, HH), lambda i: (i, 0)),
            pl.BlockSpec((OO, HH), lambda i: (0, 0)),
            pl.BlockSpec((1, OO), lambda i: (0, 0)),
            pl.BlockSpec((1, OO), lambda i: (0, 0)),
        ],
        out_specs=pl.BlockSpec((_RB, OO), lambda i: (i, 0)),
        out_shape=jax.ShapeDtypeStruct((NN, OO), F32),
    )(xc, post_v, post_g, post_b)


import numpy as np


def kernel(x, edge_index, pre_v, pre_g, pre_b, w1_v, w1_g, post_v, post_g,
           post_b):
    src = edge_index[0]
    dst = edge_index[1]

    counts = _deg_call(dst)
    counts_t = jnp.transpose(counts.reshape(NCORE, NN))

    x0, dis, xs0, xs1 = _pre_call(
        x, counts_t, pre_v, pre_g.reshape(1, HH), pre_b.reshape(1, HH))

    for i in range(NB):
        s0, s1 = _seg_call(xs0, xs1, src, dst)
        beta = jnp.full((1, 1), float(np.log(THETA_C / (i + 1) + 1.0)), F32)
        outs = _blk_call(s0, s1, xs0, xs1, x0, dis,
                         w1_v[i], w1_g[i].reshape(1, HH), beta,
                         last=(i == NB - 1))
        if i == NB - 1:
            xc = outs[0]
        else:
            xs0, xs1 = outs

    return _post_call(xc, post_v, post_g.reshape(1, OO),
                      post_b.reshape(1, OO))
